# baseline (speedup 1.0000x reference)
import functools

import jax
import jax.numpy as jnp
from jax import lax
from jax.experimental import pallas as pl
from jax.experimental.pallas import tpu as pltpu
from jax.experimental.pallas import tpu_sc as plsc

N_NODES = 10000
N_PAD = 10240
IN_F = 128
OUT_F = 128
NREL = 8

NC = 2
NS = 16
NW = NC * NS
GROUP = 128
ROWS_PT = N_PAD // NS

_MM_ROWS = 1024


def _mm_body(x_ref, w_ref, w0_ref, y_ref, init_ref):
    xb = x_ref[...].astype(jnp.bfloat16)
    for r in range(NREL):
        y_ref[r] = jnp.dot(xb, w_ref[r], preferred_element_type=jnp.float32)
    init_ref[0] = jnp.dot(xb, w0_ref[...], preferred_element_type=jnp.float32)
    init_ref[1] = jnp.zeros((_MM_ROWS, OUT_F), jnp.float32)


def _relation_matmuls(x, weight, w0):
    nblk = N_PAD // _MM_ROWS
    return pl.pallas_call(
        _mm_body,
        grid=(nblk,),
        in_specs=[
            pl.BlockSpec((_MM_ROWS, IN_F), lambda i: (i, 0)),
            pl.BlockSpec((NREL, IN_F, OUT_F), lambda i: (0, 0, 0)),
            pl.BlockSpec((IN_F, OUT_F), lambda i: (0, 0)),
        ],
        out_specs=[
            pl.BlockSpec((NREL, _MM_ROWS, OUT_F), lambda i: (0, i, 0)),
            pl.BlockSpec((2, _MM_ROWS, OUT_F), lambda i: (0, i, 0)),
        ],
        out_shape=[
            jax.ShapeDtypeStruct((NREL, N_PAD, OUT_F), jnp.float32),
            jax.ShapeDtypeStruct((2, N_PAD, OUT_F), jnp.float32),
        ],
    )(x, weight, w0)


def _make_sc_kernel(nsb, sbg):
    mesh = plsc.VectorSubcoreMesh(core_axis_name="c", subcore_axis_name="s")

    @functools.partial(
        pl.kernel,
        out_type=jax.ShapeDtypeStruct((NC, N_PAD, OUT_F), jnp.float32),
        mesh=mesh,
        scratch_types=[
            pltpu.VMEM((4, sbg, GROUP), jnp.int32),
            pltpu.VMEM((GROUP, OUT_F), jnp.float32),
            pltpu.VMEM((GROUP, OUT_F), jnp.float32),
            pltpu.VMEM_SHARED((N_PAD, OUT_F), jnp.float32),
            pltpu.SemaphoreType.DMA,
            pltpu.SemaphoreType.DMA,
            pltpu.SemaphoreType.DMA,
            pltpu.SemaphoreType.DMA,
            pltpu.SemaphoreType.DMA,
        ],
    )
    def sc_kernel(y_hbm, init_hbm, meta_hbm,
                  out_hbm, meta_v, rows0, rows1,
                  hsum, gsem0, gsem1, ssem0, ssem1, isem):
        cid = lax.axis_index("c")
        sid = lax.axis_index("s")
        wid = cid * NS + sid
        r0 = sid * ROWS_PT
        rows = (rows0, rows1)
        gsem = (gsem0, gsem1)
        ssem = (ssem0, ssem1)

        seed = pltpu.async_copy(init_hbm.at[cid, pl.ds(r0, ROWS_PT)],
                                hsum.at[pl.ds(r0, ROWS_PT)], isem)

        def scale_group(g, buf):
            @plsc.parallel_loop(0, GROUP // 16, unroll=2)
            def scale_body(eb):
                nv = lax.bitcast_convert_type(
                    meta_v[3, g, pl.ds(eb * 16, 16)], jnp.float32)
                for i in range(16):
                    s = nv[i]
                    e = eb * 16 + i
                    for j in range(OUT_F // 16):
                        sl = pl.ds(j * 16, 16)
                        buf[e, sl] = buf[e, sl] * s

        def sb_body(sb, carry):
            pltpu.sync_copy(meta_hbm.at[wid, sb], meta_v)

            @plsc.parallel_loop(0, sbg * (GROUP // 16), unroll=2)
            def fuse_body(k):
                g = k // (GROUP // 16)
                sl = pl.ds((k % (GROUP // 16)) * 16, 16)
                meta_v[0, g, sl] = meta_v[0, g, sl] + meta_v[1, g, sl] * N_PAD

            gath = {}
            scat = {}
            gath[0] = pltpu.async_copy(y_hbm.at[meta_v.at[0, 0]], rows[0], gsem[0])

            @pl.when(sb == 0)
            def _():
                seed.wait()
                plsc.subcore_barrier()

            for g in range(sbg):
                b = g % 2
                if g + 1 < sbg:
                    if g - 1 >= 0:
                        scat[g - 1].wait()
                    gath[g + 1] = pltpu.async_copy(
                        y_hbm.at[meta_v.at[0, g + 1]], rows[1 - b], gsem[1 - b])
                gath[g].wait()
                scale_group(g, rows[b])
                scat[g] = pltpu.async_copy(
                    rows[b], hsum.at[meta_v.at[2, g]], ssem[b], add=True)
            scat[sbg - 2].wait()
            scat[sbg - 1].wait()
            return carry
        lax.fori_loop(0, nsb, sb_body, 0)

        plsc.subcore_barrier()

        pltpu.sync_copy(hsum.at[pl.ds(r0, ROWS_PT)],
                        out_hbm.at[cid, pl.ds(r0, ROWS_PT)])

    return sc_kernel


def _fin_body(p_ref, o_ref):
    o_ref[...] = jnp.maximum(p_ref[0] + p_ref[1], 0.0)


def _combine(parts):
    nblk = N_NODES // 1000
    return pl.pallas_call(
        _fin_body,
        grid=(nblk,),
        in_specs=[pl.BlockSpec((NC, 1000, OUT_F), lambda i: (0, i, 0))],
        out_specs=pl.BlockSpec((1000, OUT_F), lambda i: (i, 0)),
        out_shape=jax.ShapeDtypeStruct((N_NODES, OUT_F), jnp.float32),
    )(parts)


def kernel(x, edge_index, rel_type, norm, weight, W_0):
    n_edges = edge_index.shape[1]
    sbg = 20
    sb_edges = sbg * GROUP
    nsb = -(-n_edges // (NW * sb_edges))
    e_pad = NW * nsb * sb_edges

    src = edge_index[0].astype(jnp.int32)
    dst = edge_index[1].astype(jnp.int32)
    rel = rel_type.astype(jnp.int32)
    nrm = lax.bitcast_convert_type(
        norm.reshape(-1).astype(jnp.float32), jnp.int32)

    pad = e_pad - n_edges
    shp = (NW, nsb, sbg, GROUP)
    spread = jnp.arange(pad, dtype=jnp.int32) % N_NODES
    src = jnp.concatenate([src, spread]).reshape(shp)
    dst = jnp.concatenate([dst, spread]).reshape(shp)
    rel = jnp.pad(rel, (0, pad)).reshape(shp)
    nrm = jnp.pad(nrm, (0, pad)).reshape(shp)
    meta = jnp.stack([src, rel, dst, nrm], axis=2)

    y, init = _relation_matmuls(x, weight.astype(jnp.bfloat16),
                                W_0.astype(jnp.bfloat16))
    y2 = y.reshape(NREL * N_PAD, OUT_F)

    parts = _make_sc_kernel(nsb, sbg)(y2, init, meta)
    return _combine(parts)


# speedup vs baseline: 1.0118x; 1.0118x over previous
import functools

import jax
import jax.numpy as jnp
from jax import lax
from jax.experimental import pallas as pl
from jax.experimental.pallas import tpu as pltpu
from jax.experimental.pallas import tpu_sc as plsc

N_NODES = 10000
N_PAD = 10240
IN_F = 128
OUT_F = 128
NREL = 8

NC = 2
NS = 16
NW = NC * NS
GROUP = 128
ROWS_PT = N_PAD // NS

_MM_ROWS = 1024


def _mm_body(x_ref, w_ref, w0_ref, y_ref, init_ref):
    xb = x_ref[...].astype(jnp.bfloat16)
    for r in range(NREL):
        y_ref[r] = jnp.dot(xb, w_ref[r], preferred_element_type=jnp.float32)
    init_ref[0] = jnp.dot(xb, w0_ref[...], preferred_element_type=jnp.float32)
    init_ref[1] = jnp.zeros((_MM_ROWS, OUT_F), jnp.float32)


def _relation_matmuls(x, weight, w0):
    nblk = N_PAD // _MM_ROWS
    return pl.pallas_call(
        _mm_body,
        grid=(nblk,),
        in_specs=[
            pl.BlockSpec((_MM_ROWS, IN_F), lambda i: (i, 0)),
            pl.BlockSpec((NREL, IN_F, OUT_F), lambda i: (0, 0, 0)),
            pl.BlockSpec((IN_F, OUT_F), lambda i: (0, 0)),
        ],
        out_specs=[
            pl.BlockSpec((NREL, _MM_ROWS, OUT_F), lambda i: (0, i, 0)),
            pl.BlockSpec((2, _MM_ROWS, OUT_F), lambda i: (0, i, 0)),
        ],
        out_shape=[
            jax.ShapeDtypeStruct((NREL, N_PAD, OUT_F), jnp.float32),
            jax.ShapeDtypeStruct((2, N_PAD, OUT_F), jnp.float32),
        ],
    )(x, weight, w0)


def _make_sc_kernel(nsb, sbg):
    mesh = plsc.VectorSubcoreMesh(core_axis_name="c", subcore_axis_name="s")

    @functools.partial(
        pl.kernel,
        out_type=jax.ShapeDtypeStruct((NC, N_PAD, OUT_F), jnp.float32),
        mesh=mesh,
        scratch_types=[
            pltpu.VMEM((4, sbg, GROUP), jnp.int32),
            pltpu.VMEM((GROUP, OUT_F), jnp.float32),
            pltpu.VMEM((GROUP, OUT_F), jnp.float32),
            pltpu.VMEM_SHARED((N_PAD, OUT_F), jnp.float32),
            pltpu.SemaphoreType.DMA,
            pltpu.SemaphoreType.DMA,
            pltpu.SemaphoreType.DMA,
            pltpu.SemaphoreType.DMA,
        ],
    )
    def sc_kernel(y_hbm, init_hbm, meta_hbm,
                  out_hbm, meta_v, rows0, rows1,
                  hsum, gsem0, gsem1, ssem0, ssem1):
        cid = lax.axis_index("c")
        sid = lax.axis_index("s")
        wid = cid * NS + sid
        r0 = sid * ROWS_PT
        rows = (rows0, rows1)
        gsem = (gsem0, gsem1)
        ssem = (ssem0, ssem1)

        pltpu.sync_copy(init_hbm.at[cid, pl.ds(r0, ROWS_PT)],
                        hsum.at[pl.ds(r0, ROWS_PT)])

        plsc.subcore_barrier()

        def scale_group(g, buf):
            @plsc.parallel_loop(0, GROUP // 16, unroll=2)
            def scale_body(eb):
                nv = lax.bitcast_convert_type(
                    meta_v[3, g, pl.ds(eb * 16, 16)], jnp.float32)
                for i in range(16):
                    s = nv[i]
                    e = eb * 16 + i
                    for j in range(OUT_F // 16):
                        sl = pl.ds(j * 16, 16)
                        buf[e, sl] = buf[e, sl] * s

        def sb_body(sb, carry):
            pltpu.sync_copy(meta_hbm.at[wid, sb], meta_v)

            @plsc.parallel_loop(0, sbg * (GROUP // 16), unroll=2)
            def fuse_body(k):
                g = k // (GROUP // 16)
                sl = pl.ds((k % (GROUP // 16)) * 16, 16)
                meta_v[0, g, sl] = meta_v[0, g, sl] + meta_v[1, g, sl] * N_PAD

            gath = {}
            scat = {}
            gath[0] = pltpu.async_copy(y_hbm.at[meta_v.at[0, 0]], rows[0], gsem[0])
            for g in range(sbg):
                b = g % 2
                if g + 1 < sbg:
                    if g - 1 >= 0:
                        scat[g - 1].wait()
                    gath[g + 1] = pltpu.async_copy(
                        y_hbm.at[meta_v.at[0, g + 1]], rows[1 - b], gsem[1 - b])
                gath[g].wait()
                scale_group(g, rows[b])
                scat[g] = pltpu.async_copy(
                    rows[b], hsum.at[meta_v.at[2, g]], ssem[b], add=True)
            scat[sbg - 2].wait()
            scat[sbg - 1].wait()
            return carry
        lax.fori_loop(0, nsb, sb_body, 0)

        plsc.subcore_barrier()

        pltpu.sync_copy(hsum.at[pl.ds(r0, ROWS_PT)],
                        out_hbm.at[cid, pl.ds(r0, ROWS_PT)])

    return sc_kernel


def _fin_body(p_ref, o_ref):
    o_ref[...] = jnp.maximum(p_ref[0] + p_ref[1], 0.0)


def _combine(parts):
    nblk = N_NODES // 1000
    return pl.pallas_call(
        _fin_body,
        grid=(nblk,),
        in_specs=[pl.BlockSpec((NC, 1000, OUT_F), lambda i: (0, i, 0))],
        out_specs=pl.BlockSpec((1000, OUT_F), lambda i: (i, 0)),
        out_shape=jax.ShapeDtypeStruct((N_NODES, OUT_F), jnp.float32),
    )(parts)


def kernel(x, edge_index, rel_type, norm, weight, W_0):
    n_edges = edge_index.shape[1]
    sbg = 16
    sb_edges = sbg * GROUP
    nsb = -(-n_edges // (NW * sb_edges))
    e_pad = NW * nsb * sb_edges

    src = edge_index[0].astype(jnp.int32)
    dst = edge_index[1].astype(jnp.int32)
    rel = rel_type.astype(jnp.int32)
    nrm = lax.bitcast_convert_type(
        norm.reshape(-1).astype(jnp.float32), jnp.int32)

    pad = e_pad - n_edges
    shp = (NW, nsb, sbg, GROUP)
    spread = jnp.arange(pad, dtype=jnp.int32) % N_NODES
    src = jnp.concatenate([src, spread]).reshape(shp)
    dst = jnp.concatenate([dst, spread]).reshape(shp)
    rel = jnp.pad(rel, (0, pad)).reshape(shp)
    nrm = jnp.pad(nrm, (0, pad)).reshape(shp)
    meta = jnp.stack([src, rel, dst, nrm], axis=2)

    y, init = _relation_matmuls(x, weight.astype(jnp.bfloat16),
                                W_0.astype(jnp.bfloat16))
    y2 = y.reshape(NREL * N_PAD, OUT_F)

    parts = _make_sc_kernel(nsb, sbg)(y2, init, meta)
    return _combine(parts)
